# Initial kernel scaffold; baseline (speedup 1.0000x reference)
import jax
import jax.numpy as jnp
from jax.experimental import pallas as pl


def kernel(x, w_primary, bn1_gamma, bn1_beta, bn1_mean, bn1_var, w_dw, bn2_gamma, bn2_beta, bn2_mean, bn2_var):
    raise NotImplementedError("write your pallas kernel here")


# speedup vs baseline: 2.0078x; 2.0078x over previous
import functools

import jax
import jax.numpy as jnp
from jax.experimental import pallas as pl
from jax.experimental.pallas import tpu as pltpu


def _fold_bn(w, gamma, beta, mean, var, eps=1e-5):
    scale = gamma / jnp.sqrt(var + eps)
    w_eff = w * scale.reshape((-1,) + (1,) * (w.ndim - 1))
    b_eff = beta - mean * scale
    return w_eff, b_eff


def _fused_kernel(x_ref, w1_ref, b1_ref, w2_ref, b2_ref, o_ref, xp_ref, *,
                  c1, H, W, pad):
    HW = H * W
    y1 = jnp.dot(w1_ref[...], x_ref[...], preferred_element_type=jnp.float32)
    y1 = jnp.maximum(y1 + b1_ref[...], 0.0)
    o_ref[0:c1, :] = y1.astype(o_ref.dtype)

    xp_ref[:, 0:pad] = jnp.zeros((c1, pad), jnp.float32)
    xp_ref[:, pad + HW:] = jnp.zeros((c1, pad), jnp.float32)
    xp_ref[:, pad:pad + HW] = y1

    w2 = w2_ref[...]

    def tap(dh, dw):
        off = pad + (dh - 1) * W + (dw - 1)
        return w2[:, dh * 3 + dw:dh * 3 + dw + 1] * xp_ref[:, off:off + HW]

    left = tap(0, 0) + tap(1, 0) + tap(2, 0)
    center = tap(0, 1) + tap(1, 1) + tap(2, 1)
    right = tap(0, 2) + tap(1, 2) + tap(2, 2)

    w_idx = jax.lax.broadcasted_iota(jnp.int32, (c1, HW), 1) % W
    y2 = (center
          + jnp.where(w_idx > 0, left, 0.0)
          + jnp.where(w_idx < W - 1, right, 0.0))
    y2 = jnp.maximum(y2 + b2_ref[...], 0.0)
    o_ref[c1:2 * c1, :] = y2.astype(o_ref.dtype)


def kernel(x, w_primary, bn1_gamma, bn1_beta, bn1_mean, bn1_var,
           w_dw, bn2_gamma, bn2_beta, bn2_mean, bn2_var):
    B, cin, H, W = x.shape
    HW = H * W
    c1 = w_primary.shape[0]
    pad = ((W + 1) + 127) // 128 * 128

    w1, b1 = _fold_bn(w_primary.reshape(c1, cin),
                      bn1_gamma, bn1_beta, bn1_mean, bn1_var)
    w2, b2 = _fold_bn(w_dw.reshape(c1, 9),
                      bn2_gamma, bn2_beta, bn2_mean, bn2_var)
    w1 = w1.astype(jnp.float32)
    b1 = b1.reshape(c1, 1).astype(jnp.float32)
    w2 = w2.astype(jnp.float32)
    b2 = b2.reshape(c1, 1).astype(jnp.float32)

    x3 = x.reshape(B, cin, HW)
    out = pl.pallas_call(
        functools.partial(_fused_kernel, c1=c1, H=H, W=W, pad=pad),
        out_shape=jax.ShapeDtypeStruct((B, 2 * c1, HW), x.dtype),
        grid=(B,),
        in_specs=[
            pl.BlockSpec((None, cin, HW), lambda b: (b, 0, 0)),
            pl.BlockSpec((c1, cin), lambda b: (0, 0)),
            pl.BlockSpec((c1, 1), lambda b: (0, 0)),
            pl.BlockSpec((c1, 9), lambda b: (0, 0)),
            pl.BlockSpec((c1, 1), lambda b: (0, 0)),
        ],
        out_specs=pl.BlockSpec((None, 2 * c1, HW), lambda b: (b, 0, 0)),
        scratch_shapes=[pltpu.VMEM((c1, HW + 2 * pad), jnp.float32)],
        compiler_params=pltpu.CompilerParams(
            dimension_semantics=("parallel",)),
        cost_estimate=pl.CostEstimate(
            flops=int(2 * B * HW * cin * c1 + 2 * B * c1 * HW * 9),
            transcendentals=0,
            bytes_accessed=int(4 * (B * cin * HW + B * 2 * c1 * HW))),
    )(x3, w1, b1, w2, b2)
    return out.reshape(B, 2 * c1, H, W)
